# baseline (speedup 1.0000x reference)
import functools

import jax
import jax.numpy as jnp
from jax import lax
from jax.experimental import pallas as pl
from jax.experimental.pallas import tpu as pltpu
from jax.experimental.pallas import tpu_sc as plsc

N = 10000
E = 320000
D = 128

NC = 2
NS = 16
NW = NC * NS
K = 128
CH0 = 112
CH1 = 46
CH_MAX = CH0
E_PAD = NS * K * (CH0 + CH1)
N_PAD = 10112
ROWS_PER_TILE = N_PAD // NS
L = 16

BN = 2000


def _sc_aggregate(x, packed_idx):
    mesh = plsc.VectorSubcoreMesh(core_axis_name="c", subcore_axis_name="s")

    @functools.partial(
        pl.kernel,
        out_type=jax.ShapeDtypeStruct((NC, N_PAD, D), jnp.float32),
        mesh=mesh,
        scratch_types=[
            pltpu.VMEM((CH_MAX * K,), jnp.int32),
            pltpu.VMEM((2, K), jnp.int32),
            pltpu.VMEM((K, D), jnp.float32),
            pltpu.VMEM_SHARED((N_PAD, D), jnp.float32),
        ],
    )
    def agg_kernel(x_hbm, pk_hbm, out_hbm,
                   pk_v, u_v, rows_v, acc_sh):
        c = lax.axis_index("c")
        s = lax.axis_index("s")
        wid = c * NS + s
        r0 = s * ROWS_PER_TILE
        @pl.loop(0, K)
        def _(r):
            for cc in range(0, D, L):
                rows_v[r, pl.ds(cc, L)] = jnp.zeros((L,), jnp.float32)

        for m in range(ROWS_PER_TILE // K):
            pltpu.sync_copy(rows_v, acc_sh.at[pl.ds(r0 + m * K, K)])
        _rem = ROWS_PER_TILE % K
        if _rem:
            pltpu.sync_copy(rows_v.at[pl.ds(0, _rem)],
                            acc_sh.at[pl.ds(r0 + (ROWS_PER_TILE // K) * K, _rem)])
        start = jnp.where(c == 0, s * (CH0 * K), NS * CH0 * K + s * (CH1 * K))
        pltpu.sync_copy(pk_hbm.at[pl.ds(start, CH_MAX * K)], pk_v)
        plsc.subcore_barrier()

        nch = jnp.where(c == 0, CH0, CH1)

        @pl.loop(0, nch)
        def _(j):
            for kk in range(0, K, L):
                v = pk_v[pl.ds(j * K + kk, L)]
                u_v[0, pl.ds(kk, L)] = v & 0xFFFF
                u_v[1, pl.ds(kk, L)] = lax.shift_right_logical(v, 16)
            pltpu.sync_copy(x_hbm.at[u_v.at[0]], rows_v)
            pltpu.sync_copy(rows_v, acc_sh.at[u_v.at[1]], add=True)

        plsc.subcore_barrier()
        pltpu.sync_copy(acc_sh.at[pl.ds(r0, ROWS_PER_TILE)],
                        out_hbm.at[c, pl.ds(r0, ROWS_PER_TILE)])

    return agg_kernel(x, packed_idx)


def _mlp_body(x_ref, p_ref, w1_ref, b1_ref, w2_ref, b2_ref, o_ref):
    h = x_ref[...] + p_ref[0] + p_ref[1]
    h = jnp.maximum(
        jnp.dot(h, w1_ref[...], preferred_element_type=jnp.float32) + b1_ref[...],
        0.0)
    h = jnp.dot(h, w2_ref[...], preferred_element_type=jnp.float32) + b2_ref[...]
    o_ref[...] = jnp.maximum(h, 0.0)


def _mlp(x, p, W1, b1, W2, b2):
    return pl.pallas_call(
        _mlp_body,
        grid=(N // BN,),
        in_specs=[
            pl.BlockSpec((BN, D), lambda i: (i, 0)),
            pl.BlockSpec((NC, BN, D), lambda i: (0, i, 0)),
            pl.BlockSpec((D, D), lambda i: (0, 0)),
            pl.BlockSpec((1, D), lambda i: (0, 0)),
            pl.BlockSpec((D, D), lambda i: (0, 0)),
            pl.BlockSpec((1, D), lambda i: (0, 0)),
        ],
        out_specs=pl.BlockSpec((BN, D), lambda i: (i, 0)),
        out_shape=jax.ShapeDtypeStruct((N, D), jnp.float32),
    )(x, p, W1, b1.reshape(1, D), W2, b2.reshape(1, D))


PB = 6400
PBLK_E = E // PB


def _pack_body(s_ref, d_ref, o_ref):
    i = pl.program_id(0)
    e0 = (i * PB
          + lax.broadcasted_iota(jnp.int32, (1, 8, PB // 8), 1) * (PB // 8)
          + lax.broadcasted_iota(jnp.int32, (1, 8, PB // 8), 2))
    v = s_ref[...] | (d_ref[...] << 16)
    o_ref[...] = jnp.where(e0 < E, v, N << 16)


def _pack(edge_index, nblocks):
    clamp = lambda i: (jnp.minimum(i, PBLK_E - 1), 0, 0)
    shape3 = (PBLK_E, 8, PB // 8)
    blk = (1, 8, PB // 8)
    out = pl.pallas_call(
        _pack_body,
        grid=(nblocks,),
        in_specs=[pl.BlockSpec(blk, clamp), pl.BlockSpec(blk, clamp)],
        out_specs=pl.BlockSpec(blk, lambda i: (i, 0, 0)),
        out_shape=jax.ShapeDtypeStruct((nblocks, 8, PB // 8), jnp.int32),
    )(edge_index[0].reshape(shape3), edge_index[1].reshape(shape3))
    return out.reshape(-1)


def kernel(x, edge_index, W1, b1, W2, b2):
    total = NS * CH0 * K + NS * CH1 * K + (CH_MAX - CH1) * K
    nblocks = -(-total // PB)
    packed = _pack(edge_index, nblocks)
    p = _sc_aggregate(x, packed)
    return _mlp(x, p, W1, b1, W2, b2)


# speedup vs baseline: 1.1521x; 1.1521x over previous
import functools

import jax
import jax.numpy as jnp
from jax import lax
from jax.experimental import pallas as pl
from jax.experimental.pallas import tpu as pltpu
from jax.experimental.pallas import tpu_sc as plsc

N = 10000
E = 320000
D = 128

NC = 2
NS = 16
NW = NC * NS
K = 128
CH0 = 112
CH1 = 46
CH_MAX = CH0
E_PAD = NS * K * (CH0 + CH1)
N_PAD = 10112
ROWS_PER_TILE = N_PAD // NS
L = 16

BN = 2000


def _sc_aggregate(x, packed_idx):
    mesh = plsc.VectorSubcoreMesh(core_axis_name="c", subcore_axis_name="s")

    @functools.partial(
        pl.kernel,
        out_type=jax.ShapeDtypeStruct((NC, N_PAD, D), jnp.float32),
        mesh=mesh,
        scratch_types=[
            pltpu.VMEM((CH_MAX * K,), jnp.int32),
            pltpu.VMEM((2, K), jnp.int32),
            pltpu.VMEM((K, D), jnp.float32),
            pltpu.VMEM_SHARED((N_PAD, D), jnp.float32),
        ],
    )
    def agg_kernel(x_hbm, pk_hbm, out_hbm,
                   pk_v, u_v, rows_v, acc_sh):
        c = lax.axis_index("c")
        s = lax.axis_index("s")
        wid = c * NS + s
        r0 = s * ROWS_PER_TILE
        @pl.loop(0, K)
        def _(r):
            for cc in range(0, D, L):
                rows_v[r, pl.ds(cc, L)] = jnp.zeros((L,), jnp.float32)

        for m in range(ROWS_PER_TILE // K):
            pltpu.sync_copy(rows_v, acc_sh.at[pl.ds(r0 + m * K, K)])
        _rem = ROWS_PER_TILE % K
        if _rem:
            pltpu.sync_copy(rows_v.at[pl.ds(0, _rem)],
                            acc_sh.at[pl.ds(r0 + (ROWS_PER_TILE // K) * K, _rem)])
        start = jnp.where(c == 0, s * (CH0 * K), NS * CH0 * K + s * (CH1 * K))
        pltpu.sync_copy(pk_hbm.at[pl.ds(start, CH_MAX * K)], pk_v)
        plsc.subcore_barrier()

        nch = jnp.where(c == 0, CH0, CH1)

        @pl.loop(0, nch)
        def _(j):
            for kk in range(0, K, L):
                v = pk_v[pl.ds(j * K + kk, L)]
                u_v[0, pl.ds(kk, L)] = v & 0xFFFF
                u_v[1, pl.ds(kk, L)] = lax.shift_right_logical(v, 16)
            pltpu.sync_copy(x_hbm.at[u_v.at[0]], rows_v)
            pltpu.sync_copy(rows_v, acc_sh.at[u_v.at[1]], add=True)

        plsc.subcore_barrier()
        pltpu.sync_copy(acc_sh.at[pl.ds(r0, ROWS_PER_TILE)],
                        out_hbm.at[c, pl.ds(r0, ROWS_PER_TILE)])

    return agg_kernel(x, packed_idx)


def _mlp_body(x_ref, p_ref, w1_ref, b1_ref, w2_ref, b2_ref, o_ref):
    h = x_ref[...] + p_ref[0] + p_ref[1]
    h = jnp.maximum(
        jnp.dot(h, w1_ref[...], preferred_element_type=jnp.float32) + b1_ref[...],
        0.0)
    h = jnp.dot(h, w2_ref[...], preferred_element_type=jnp.float32) + b2_ref[...]
    o_ref[...] = jnp.maximum(h, 0.0)


def _mlp(x, p, W1, b1, W2, b2):
    return pl.pallas_call(
        _mlp_body,
        grid=(N // BN,),
        in_specs=[
            pl.BlockSpec((BN, D), lambda i: (i, 0)),
            pl.BlockSpec((NC, BN, D), lambda i: (0, i, 0)),
            pl.BlockSpec((D, D), lambda i: (0, 0)),
            pl.BlockSpec((1, D), lambda i: (0, 0)),
            pl.BlockSpec((D, D), lambda i: (0, 0)),
            pl.BlockSpec((1, D), lambda i: (0, 0)),
        ],
        out_specs=pl.BlockSpec((BN, D), lambda i: (i, 0)),
        out_shape=jax.ShapeDtypeStruct((N, D), jnp.float32),
    )(x, p, W1, b1.reshape(1, D), W2, b2.reshape(1, D))


TOTAL = NS * CH0 * K + NS * CH1 * K + (CH_MAX - CH1) * K


def _pack_body(ei_ref, o_ref):
    o_ref[pl.ds(0, E)] = ei_ref[0] | (ei_ref[1] << 16)
    o_ref[pl.ds(E, TOTAL - E)] = jnp.full((TOTAL - E,), N << 16, jnp.int32)


def _pack(edge_index):
    return pl.pallas_call(
        _pack_body,
        out_shape=jax.ShapeDtypeStruct((TOTAL,), jnp.int32),
    )(edge_index)


def kernel(x, edge_index, W1, b1, W2, b2):
    packed = _pack(edge_index)
    p = _sc_aggregate(x, packed)
    return _mlp(x, p, W1, b1, W2, b2)
